# baseline (speedup 1.0000x reference)
import functools

import jax
import jax.numpy as jnp
from jax import lax
from jax.experimental import pallas as pl
from jax.experimental.pallas import tpu as pltpu
from jax.experimental.pallas import tpu_sc as plsc

NC = 2
NS = 16
LANES = 16


@functools.lru_cache(maxsize=None)
def _make_pool_kernel(B, L, E):
    NW = NC * NS
    RPW = B // NW
    ACC = 40
    NSTR = L // ACC
    NCH = E // LANES
    assert L % ACC == 0 and ACC % 8 == 0

    mesh = plsc.VectorSubcoreMesh(core_axis_name="c", subcore_axis_name="s")

    @functools.partial(
        pl.kernel,
        mesh=mesh,
        out_type=jax.ShapeDtypeStruct((B, E), jnp.float32),
        scratch_types=[
            pltpu.VMEM((RPW * L,), jnp.int32),
            pltpu.VMEM((ACC, E), jnp.float32),
            pltpu.VMEM((ACC, E), jnp.float32),
            pltpu.VMEM((RPW, E), jnp.float32),
            pltpu.SemaphoreType.DMA,
            pltpu.SemaphoreType.DMA,
        ],
    )
    def pool(emb_hbm, ng_hbm, out_hbm, idx_v, acc0, acc1, out_v, sem0, sem1):
        cid = lax.axis_index("c")
        sid = lax.axis_index("s")
        wid = sid * NC + cid
        base = wid * RPW
        pltpu.sync_copy(ng_hbm.at[pl.ds(base * L, RPW * L)], idx_v)

        zero = jnp.zeros((LANES,), jnp.float32)
        for acc in (acc0, acc1):
            for r in range(ACC):
                for c in range(NCH):
                    acc[r, pl.ds(c * LANES, LANES)] = zero

        def fire(b, acc, sem):
            return [pltpu.async_copy(
                        emb_hbm.at[idx_v.at[pl.ds(b * L + k * ACC, ACC)]],
                        acc, sem, add=True)
                    for k in range(NSTR)]

        def wait_acc(acc, sem):
            for _ in range(NSTR):
                pltpu.make_async_copy(emb_hbm.at[pl.ds(0, ACC)], acc,
                                      sem).wait()

        def fold_and_rezero(acc, row):
            accs = tuple(acc[0, pl.ds(c * LANES, LANES)] for c in range(NCH))
            for r in range(1, ACC):
                accs = tuple(accs[c] + acc[r, pl.ds(c * LANES, LANES)]
                             for c in range(NCH))
            for r in range(ACC):
                for c in range(NCH):
                    acc[r, pl.ds(c * LANES, LANES)] = zero
            for c in range(NCH):
                out_v[row, pl.ds(c * LANES, LANES)] = accs[c]

        fire(0, acc0, sem0)

        def pair(p, carry):
            b0 = 2 * p
            fire(b0 + 1, acc1, sem1)
            wait_acc(acc0, sem0)
            fold_and_rezero(acc0, b0)
            nxt = jnp.where(b0 + 2 < RPW, b0 + 2, 0)
            fire(nxt, acc0, sem0)
            wait_acc(acc1, sem1)
            fold_and_rezero(acc1, b0 + 1)
            return carry

        lax.fori_loop(0, RPW // 2, pair, 0)
        wait_acc(acc0, sem0)
        pltpu.sync_copy(out_v, out_hbm.at[pl.ds(base, RPW)])

    return pool


def _mlp_body(nvalid, x_ref, w1_ref, b1_ref, w2_ref, b2_ref, w3_ref,
              b3_ref, tgt_ref, out_ref, loss_ref):
    i = pl.program_id(0)
    x = x_ref[...]
    h1 = jnp.maximum(
        jnp.dot(x, w1_ref[...], preferred_element_type=jnp.float32)
        + b1_ref[...], 0.0)
    h2 = jnp.maximum(
        jnp.dot(h1, w2_ref[...], preferred_element_type=jnp.float32)
        + b2_ref[...], 0.0)
    logits = (jnp.dot(h2, w3_ref[...], preferred_element_type=jnp.float32)
              + b3_ref[...])
    m1 = jnp.max(logits, axis=1, keepdims=True)
    lse1 = jnp.log(jnp.sum(jnp.exp(logits - m1), axis=1, keepdims=True)) + m1
    outs = logits - lse1
    out_ref[...] = outs
    logp = outs
    cols = lax.broadcasted_iota(jnp.int32, logp.shape, 1)
    sel = jnp.where(cols == tgt_ref[...], logp, 0.0)
    nll_sum = -jnp.sum(sel)

    @pl.when(i == 0)
    def _():
        loss_ref[...] = jnp.zeros((1, 1), jnp.float32)

    loss_ref[...] += (nll_sum * (1.0 / nvalid)).reshape(1, 1)


def _mlp_tc(pooled, W1, b1, W2, b2, W3, b3, target, nvalid):
    B, E = pooled.shape
    H1 = W1.shape[1]
    H2 = W2.shape[1]
    NCLS_ = W3.shape[1]
    PAD = 128
    BLK = min(B, 4096)

    W2p = jnp.pad(W2, ((0, 0), (0, PAD - H2)))
    b2p = jnp.pad(b2, (0, PAD - H2)).reshape(1, PAD)
    W3p = jnp.pad(W3, ((0, PAD - H2), (0, PAD - NCLS_)))
    b3p = jnp.pad(b3, (0, PAD - NCLS_),
                  constant_values=-1e30).reshape(1, PAD)
    b1r = b1.reshape(1, H1)
    tgt2 = target.astype(jnp.int32).reshape(B, 1)

    grid = (B // BLK,)
    out_p, loss = pl.pallas_call(
        functools.partial(_mlp_body, float(nvalid)),
        grid=grid,
        in_specs=[
            pl.BlockSpec((BLK, E), lambda i: (i, 0)),
            pl.BlockSpec((E, H1), lambda i: (0, 0)),
            pl.BlockSpec((1, H1), lambda i: (0, 0)),
            pl.BlockSpec((H1, PAD), lambda i: (0, 0)),
            pl.BlockSpec((1, PAD), lambda i: (0, 0)),
            pl.BlockSpec((PAD, PAD), lambda i: (0, 0)),
            pl.BlockSpec((1, PAD), lambda i: (0, 0)),
            pl.BlockSpec((BLK, 1), lambda i: (i, 0)),
        ],
        out_specs=[
            pl.BlockSpec((BLK, PAD), lambda i: (i, 0)),
            pl.BlockSpec((1, 1), lambda i: (0, 0)),
        ],
        out_shape=[
            jax.ShapeDtypeStruct((B, PAD), jnp.float32),
            jax.ShapeDtypeStruct((1, 1), jnp.float32),
        ],
    )(pooled, W1, b1r, W2p, b2p, W3p, b3p, tgt2)
    return out_p[:, :NCLS_], loss[0, 0]


def kernel(ngrams, num_ngram, target, emb, W1, b1, W2, b2, W3, b3):
    del num_ngram
    ngrams = ngrams.astype(jnp.int32)
    B, L = ngrams.shape
    V, E = emb.shape
    NSPLIT = 1
    CB = B // NSPLIT
    pool = _make_pool_kernel(CB, L, E)
    ng_flat = ngrams.reshape(B * L)
    outs, losses = [], []
    for i in range(NSPLIT):
        pooled = pool(emb, lax.slice_in_dim(ng_flat, i * CB * L,
                                            (i + 1) * CB * L))
        o, l = _mlp_tc(pooled, W1, b1, W2, b2, W3, b3,
                       lax.slice_in_dim(target, i * CB, (i + 1) * CB), B)
        outs.append(o)
        losses.append(l)
    outputs = jnp.concatenate(outs, axis=0)
    loss = sum(losses[1:], losses[0])
    return (outputs, loss, target)


# speedup vs baseline: 1.2061x; 1.2061x over previous
import functools

import jax
import jax.numpy as jnp
from jax import lax
from jax.experimental import pallas as pl
from jax.experimental.pallas import tpu as pltpu
from jax.experimental.pallas import tpu_sc as plsc

NC = 2
NS = 16
LANES = 16


@functools.lru_cache(maxsize=None)
def _make_pool_kernel(B, L, E):
    NW = NC * NS
    RPW = B // NW
    C1 = min(L, 128)
    C2 = L - C1
    NCH = E // LANES
    UNR = 8

    mesh = plsc.VectorSubcoreMesh(core_axis_name="c", subcore_axis_name="s")

    @functools.partial(
        pl.kernel,
        mesh=mesh,
        out_type=jax.ShapeDtypeStruct((B, E), jnp.float32),
        scratch_types=[
            pltpu.VMEM((RPW, L), jnp.int32),
            pltpu.VMEM((L, E), jnp.float32),
            pltpu.VMEM((L, E), jnp.float32),
            pltpu.VMEM((RPW, E), jnp.float32),
            pltpu.SemaphoreType.DMA,
            pltpu.SemaphoreType.DMA,
        ],
    )
    def pool(emb_hbm, ng_hbm, out_hbm, idx_v, buf0, buf1, out_v, sem0, sem1):
        cid = lax.axis_index("c")
        sid = lax.axis_index("s")
        wid = sid * NC + cid
        base = wid * RPW
        pltpu.sync_copy(ng_hbm.at[pl.ds(base, RPW)], idx_v)

        def fire(b, buf, sem):
            pltpu.async_copy(emb_hbm.at[idx_v.at[b, pl.ds(0, C1)]],
                             buf.at[pl.ds(0, C1)], sem)
            if C2:
                pltpu.async_copy(emb_hbm.at[idx_v.at[b, pl.ds(C1, C2)]],
                                 buf.at[pl.ds(C1, C2)], sem)

        def wait_gather(buf, sem):
            pltpu.make_async_copy(emb_hbm.at[pl.ds(0, L)], buf, sem).wait()

        def reduce_into(buf, row):
            def body(r, accs):
                rr = UNR * r
                for k in range(UNR):
                    accs = tuple(
                        accs[c] + buf[rr + k, pl.ds(c * LANES, LANES)]
                        for c in range(NCH))
                return accs

            init = tuple(jnp.zeros((LANES,), jnp.float32) for _ in range(NCH))
            accs = lax.fori_loop(0, L // UNR, body, init)
            for k in range(L % UNR):
                accs = tuple(
                    accs[c] + buf[L - (L % UNR) + k,
                                  pl.ds(c * LANES, LANES)]
                    for c in range(NCH))
            for c in range(NCH):
                out_v[row, pl.ds(c * LANES, LANES)] = accs[c]

        fire(0, buf0, sem0)

        def pair(p, carry):
            b0 = 2 * p
            fire(b0 + 1, buf1, sem1)
            wait_gather(buf0, sem0)
            reduce_into(buf0, b0)
            nxt = jnp.where(b0 + 2 < RPW, b0 + 2, 0)
            fire(nxt, buf0, sem0)
            wait_gather(buf1, sem1)
            reduce_into(buf1, b0 + 1)
            return carry

        lax.fori_loop(0, RPW // 2, pair, 0)
        wait_gather(buf0, sem0)
        pltpu.sync_copy(out_v, out_hbm.at[pl.ds(base, RPW)])

    return pool


def _mlp_body(nvalid, x_ref, w1_ref, b1_ref, w2_ref, b2_ref, w3_ref,
              b3_ref, tgt_ref, out_ref, loss_ref):
    i = pl.program_id(0)
    x = x_ref[...]
    h1 = jnp.maximum(
        jnp.dot(x, w1_ref[...], preferred_element_type=jnp.float32)
        + b1_ref[...], 0.0)
    h2 = jnp.maximum(
        jnp.dot(h1, w2_ref[...], preferred_element_type=jnp.float32)
        + b2_ref[...], 0.0)
    logits = (jnp.dot(h2, w3_ref[...], preferred_element_type=jnp.float32)
              + b3_ref[...])
    m1 = jnp.max(logits, axis=1, keepdims=True)
    lse1 = jnp.log(jnp.sum(jnp.exp(logits - m1), axis=1, keepdims=True)) + m1
    outs = logits - lse1
    out_ref[...] = outs
    logp = outs
    cols = lax.broadcasted_iota(jnp.int32, logp.shape, 1)
    sel = jnp.where(cols == tgt_ref[...], logp, 0.0)
    nll_sum = -jnp.sum(sel)

    @pl.when(i == 0)
    def _():
        loss_ref[...] = jnp.zeros((1, 1), jnp.float32)

    loss_ref[...] += (nll_sum * (1.0 / nvalid)).reshape(1, 1)


def _mlp_tc(pooled, W1, b1, W2, b2, W3, b3, target, nvalid):
    B, E = pooled.shape
    H1 = W1.shape[1]
    H2 = W2.shape[1]
    NCLS_ = W3.shape[1]
    PAD = 128
    BLK = min(B, 4096)

    W2p = jnp.pad(W2, ((0, 0), (0, PAD - H2)))
    b2p = jnp.pad(b2, (0, PAD - H2)).reshape(1, PAD)
    W3p = jnp.pad(W3, ((0, PAD - H2), (0, PAD - NCLS_)))
    b3p = jnp.pad(b3, (0, PAD - NCLS_),
                  constant_values=-1e30).reshape(1, PAD)
    b1r = b1.reshape(1, H1)
    tgt2 = target.astype(jnp.int32).reshape(B, 1)

    grid = (B // BLK,)
    out_p, loss = pl.pallas_call(
        functools.partial(_mlp_body, float(nvalid)),
        grid=grid,
        in_specs=[
            pl.BlockSpec((BLK, E), lambda i: (i, 0)),
            pl.BlockSpec((E, H1), lambda i: (0, 0)),
            pl.BlockSpec((1, H1), lambda i: (0, 0)),
            pl.BlockSpec((H1, PAD), lambda i: (0, 0)),
            pl.BlockSpec((1, PAD), lambda i: (0, 0)),
            pl.BlockSpec((PAD, PAD), lambda i: (0, 0)),
            pl.BlockSpec((1, PAD), lambda i: (0, 0)),
            pl.BlockSpec((BLK, 1), lambda i: (i, 0)),
        ],
        out_specs=[
            pl.BlockSpec((BLK, PAD), lambda i: (i, 0)),
            pl.BlockSpec((1, 1), lambda i: (0, 0)),
        ],
        out_shape=[
            jax.ShapeDtypeStruct((B, PAD), jnp.float32),
            jax.ShapeDtypeStruct((1, 1), jnp.float32),
        ],
    )(pooled, W1, b1r, W2p, b2p, W3p, b3p, tgt2)
    return out_p[:, :NCLS_], loss[0, 0]


def kernel(ngrams, num_ngram, target, emb, W1, b1, W2, b2, W3, b3):
    del num_ngram
    ngrams = ngrams.astype(jnp.int32)
    B, L = ngrams.shape
    V, E = emb.shape
    pooled = _make_pool_kernel(B, L, E)(emb, ngrams)
    outputs, loss = _mlp_tc(pooled, W1, b1, W2, b2, W3, b3, target, B)
    return (outputs, loss, target)


# speedup vs baseline: 1.4796x; 1.2268x over previous
import functools

import jax
import jax.numpy as jnp
from jax import lax
from jax.experimental import pallas as pl
from jax.experimental.pallas import tpu as pltpu
from jax.experimental.pallas import tpu_sc as plsc

NC = 2
NS = 16
LANES = 16


@functools.lru_cache(maxsize=None)
def _make_pool_kernel(B, L, E):
    NW = NC * NS
    RPW = B // NW
    C1 = min(L, 128)
    C2 = L - C1
    NCH = E // LANES
    UNR = 8

    mesh = plsc.VectorSubcoreMesh(core_axis_name="c", subcore_axis_name="s")

    @functools.partial(
        pl.kernel,
        mesh=mesh,
        out_type=jax.ShapeDtypeStruct((B, E), jnp.float32),
        scratch_types=[
            pltpu.VMEM((RPW, L), jnp.int32),
            pltpu.VMEM((L, E), jnp.float32),
            pltpu.VMEM((L, E), jnp.float32),
            pltpu.VMEM((L, E), jnp.float32),
            pltpu.VMEM((RPW, E), jnp.float32),
            pltpu.SemaphoreType.DMA,
            pltpu.SemaphoreType.DMA,
            pltpu.SemaphoreType.DMA,
        ],
    )
    def pool(emb_hbm, ng_hbm, out_hbm, idx_v, buf0, buf1, buf2, out_v,
             sem0, sem1, sem2):
        cid = lax.axis_index("c")
        sid = lax.axis_index("s")
        wid = sid * NC + cid
        base = wid * RPW
        pltpu.sync_copy(ng_hbm.at[pl.ds(base, RPW)], idx_v)

        def fire(b, buf, sem):
            pltpu.async_copy(emb_hbm.at[idx_v.at[b, pl.ds(0, C1)]],
                             buf.at[pl.ds(0, C1)], sem)
            if C2:
                pltpu.async_copy(emb_hbm.at[idx_v.at[b, pl.ds(C1, C2)]],
                                 buf.at[pl.ds(C1, C2)], sem)

        def wait_gather(buf, sem):
            pltpu.make_async_copy(emb_hbm.at[pl.ds(0, L)], buf, sem).wait()

        def reduce_into(buf, row):
            def body(r, accs):
                rr = UNR * r
                for k in range(UNR):
                    accs = tuple(
                        accs[c] + buf[rr + k, pl.ds(c * LANES, LANES)]
                        for c in range(NCH))
                return accs

            init = tuple(jnp.zeros((LANES,), jnp.float32) for _ in range(NCH))
            accs = lax.fori_loop(0, L // UNR, body, init)
            for k in range(L % UNR):
                accs = tuple(
                    accs[c] + buf[L - (L % UNR) + k,
                                  pl.ds(c * LANES, LANES)]
                    for c in range(NCH))
            for c in range(NCH):
                out_v[row, pl.ds(c * LANES, LANES)] = accs[c]

        NTRI = (RPW - 2) // 3
        assert RPW == 3 * NTRI + 2
        fire(0, buf0, sem0)
        fire(1, buf1, sem1)

        def tri(p, carry):
            r = 3 * p
            fire(r + 2, buf2, sem2)
            wait_gather(buf0, sem0)
            reduce_into(buf0, r)
            fire(r + 3, buf0, sem0)
            wait_gather(buf1, sem1)
            reduce_into(buf1, r + 1)
            fire(r + 4, buf1, sem1)
            wait_gather(buf2, sem2)
            reduce_into(buf2, r + 2)
            return carry

        lax.fori_loop(0, NTRI, tri, 0)
        wait_gather(buf0, sem0)
        reduce_into(buf0, RPW - 2)
        wait_gather(buf1, sem1)
        reduce_into(buf1, RPW - 1)
        pltpu.sync_copy(out_v, out_hbm.at[pl.ds(base, RPW)])

    return pool


def _mlp_body(nvalid, x_ref, w1_ref, b1_ref, w2_ref, b2_ref, w3_ref,
              b3_ref, tgt_ref, out_ref, loss_ref):
    i = pl.program_id(0)
    x = x_ref[...]
    h1 = jnp.maximum(
        jnp.dot(x, w1_ref[...], preferred_element_type=jnp.float32)
        + b1_ref[...], 0.0)
    h2 = jnp.maximum(
        jnp.dot(h1, w2_ref[...], preferred_element_type=jnp.float32)
        + b2_ref[...], 0.0)
    logits = (jnp.dot(h2, w3_ref[...], preferred_element_type=jnp.float32)
              + b3_ref[...])
    m1 = jnp.max(logits, axis=1, keepdims=True)
    lse1 = jnp.log(jnp.sum(jnp.exp(logits - m1), axis=1, keepdims=True)) + m1
    outs = logits - lse1
    out_ref[...] = outs
    logp = outs
    cols = lax.broadcasted_iota(jnp.int32, logp.shape, 1)
    sel = jnp.where(cols == tgt_ref[...], logp, 0.0)
    nll_sum = -jnp.sum(sel)

    @pl.when(i == 0)
    def _():
        loss_ref[...] = jnp.zeros((1, 1), jnp.float32)

    loss_ref[...] += (nll_sum * (1.0 / nvalid)).reshape(1, 1)


def _mlp_tc(pooled, W1, b1, W2, b2, W3, b3, target, nvalid):
    B, E = pooled.shape
    H1 = W1.shape[1]
    H2 = W2.shape[1]
    NCLS_ = W3.shape[1]
    PAD = 128
    BLK = min(B, 4096)

    W2p = jnp.pad(W2, ((0, 0), (0, PAD - H2)))
    b2p = jnp.pad(b2, (0, PAD - H2)).reshape(1, PAD)
    W3p = jnp.pad(W3, ((0, PAD - H2), (0, PAD - NCLS_)))
    b3p = jnp.pad(b3, (0, PAD - NCLS_),
                  constant_values=-1e30).reshape(1, PAD)
    b1r = b1.reshape(1, H1)
    tgt2 = target.astype(jnp.int32).reshape(B, 1)

    grid = (B // BLK,)
    out_p, loss = pl.pallas_call(
        functools.partial(_mlp_body, float(nvalid)),
        grid=grid,
        in_specs=[
            pl.BlockSpec((BLK, E), lambda i: (i, 0)),
            pl.BlockSpec((E, H1), lambda i: (0, 0)),
            pl.BlockSpec((1, H1), lambda i: (0, 0)),
            pl.BlockSpec((H1, PAD), lambda i: (0, 0)),
            pl.BlockSpec((1, PAD), lambda i: (0, 0)),
            pl.BlockSpec((PAD, PAD), lambda i: (0, 0)),
            pl.BlockSpec((1, PAD), lambda i: (0, 0)),
            pl.BlockSpec((BLK, 1), lambda i: (i, 0)),
        ],
        out_specs=[
            pl.BlockSpec((BLK, PAD), lambda i: (i, 0)),
            pl.BlockSpec((1, 1), lambda i: (0, 0)),
        ],
        out_shape=[
            jax.ShapeDtypeStruct((B, PAD), jnp.float32),
            jax.ShapeDtypeStruct((1, 1), jnp.float32),
        ],
    )(pooled, W1, b1r, W2p, b2p, W3p, b3p, tgt2)
    return out_p[:, :NCLS_], loss[0, 0]


def kernel(ngrams, num_ngram, target, emb, W1, b1, W2, b2, W3, b3):
    del num_ngram
    ngrams = ngrams.astype(jnp.int32)
    B, L = ngrams.shape
    V, E = emb.shape
    pooled = _make_pool_kernel(B, L, E)(emb, ngrams)
    outputs, loss = _mlp_tc(pooled, W1, b1, W2, b2, W3, b3, target, B)
    return (outputs, loss, target)
